# baseline (speedup 1.0000x reference)
import functools

import jax
import jax.numpy as jnp
from jax import lax
from jax.experimental import pallas as pl
from jax.experimental.pallas import tpu as pltpu
from jax.experimental.pallas import tpu_sc as plsc

B = 16384
D = 128
V = 10
NC = 2
NS = 16
NW = NC * NS

B_SC = 4096
B_TC = B - B_SC
B_PER_W = B_SC // NW
NCHUNK = 2
CH = B_PER_W // NCHUNK

BLK = 1024
NB = B_TC // BLK


def _sc_lookup(table, idx_sc):
    mesh = plsc.VectorSubcoreMesh(core_axis_name="c", subcore_axis_name="s")

    @functools.partial(
        pl.kernel,
        mesh=mesh,
        out_type=jax.ShapeDtypeStruct((B_SC, D), jnp.float32),
        scratch_types=[
            pltpu.VMEM((B_PER_W,), jnp.int32),
            pltpu.VMEM_SHARED((V, D), jnp.float32),
            pltpu.VMEM((B_PER_W, D), jnp.float32),
            pltpu.SemaphoreType.DMA,
            pltpu.SemaphoreType.DMA,
        ],
    )
    def lookup_kernel(table_hbm, idx_hbm, out_hbm, idx_v, tbl_sh, rows_v,
                      gsem, wsem):
        sid = lax.axis_index("s")
        wid = sid * NC + lax.axis_index("c")
        base = wid * B_PER_W

        idx_cp = pltpu.async_copy(idx_hbm.at[pl.ds(base, B_PER_W)], idx_v,
                                  wsem)

        @pl.when(sid == 0)
        def _():
            pltpu.sync_copy(table_hbm, tbl_sh)

        idx_cp.wait()
        plsc.subcore_barrier()

        gathers = [
            pltpu.async_copy(
                tbl_sh.at[idx_v.at[pl.ds(k * CH, CH)]],
                rows_v.at[pl.ds(k * CH, CH)],
                gsem,
            )
            for k in range(NCHUNK)
        ]
        writes = []
        for k in range(NCHUNK):
            gathers[k].wait()
            writes.append(
                pltpu.async_copy(
                    rows_v.at[pl.ds(k * CH, CH)],
                    out_hbm.at[pl.ds(base + k * CH, CH)],
                    wsem,
                )
            )
        for w in writes:
            w.wait()

    return lookup_kernel(table, idx_sc)


def _tc_body(idx_ref, tbl_ref, out_ref):
    idxb = idx_ref[0]
    acc = jnp.zeros((BLK, D), jnp.float32)
    for r in range(V):
        acc = jnp.where(idxb == r, tbl_ref[r:r + 1, :], acc)
    out_ref[...] = acc


def _tc_lookup(table, idx_tc):
    idx3 = idx_tc.reshape(NB, BLK, 1)
    return pl.pallas_call(
        _tc_body,
        grid=(NB,),
        in_specs=[
            pl.BlockSpec((1, BLK, 1), lambda i: (i, 0, 0)),
            pl.BlockSpec((V, D), lambda i: (0, 0)),
        ],
        out_specs=pl.BlockSpec((BLK, D), lambda i: (i, 0)),
        out_shape=jax.ShapeDtypeStruct((B_TC, D), jnp.float32),
    )(idx3, table)


@jax.jit
def kernel(indices, table):
    idx = indices.astype(jnp.int32)
    sc_out = _sc_lookup(table, idx[:B_SC])
    tc_out = _tc_lookup(table, idx[B_SC:])
    return jnp.concatenate([sc_out, tc_out], axis=0)


# speedup vs baseline: 2.3425x; 2.3425x over previous
import functools

import jax
import jax.numpy as jnp
from jax import lax
from jax.experimental import pallas as pl
from jax.experimental.pallas import tpu as pltpu
from jax.experimental.pallas import tpu_sc as plsc

B = 16384
D = 128
V = 10
NC = 2
NS = 16
NW = NC * NS
B_PER_W = B // NW
NCHUNK = 16
CH = B_PER_W // NCHUNK


@jax.jit
def kernel(indices, table):
    mesh = plsc.VectorSubcoreMesh(core_axis_name="c", subcore_axis_name="s")

    @functools.partial(
        pl.kernel,
        mesh=mesh,
        out_type=jax.ShapeDtypeStruct((B, D), jnp.float32),
        scratch_types=[
            pltpu.VMEM((B_PER_W,), jnp.int32),
            pltpu.VMEM_SHARED((V, D), jnp.float32),
            pltpu.VMEM((B_PER_W, D), jnp.float32),
            pltpu.SemaphoreType.DMA,
            pltpu.SemaphoreType.DMA,
        ],
    )
    def lookup_kernel(table_hbm, idx_hbm, out_hbm, idx_v, tbl_sh, rows_v,
                      gsem, wsem):
        sid = lax.axis_index("s")
        wid = sid * NC + lax.axis_index("c")
        base = wid * B_PER_W

        idx_cp = pltpu.async_copy(idx_hbm.at[pl.ds(base, B_PER_W)], idx_v,
                                  wsem)

        @pl.when(sid == 0)
        def _():
            pltpu.sync_copy(table_hbm, tbl_sh)

        idx_cp.wait()
        plsc.subcore_barrier()

        gathers = [
            pltpu.async_copy(
                tbl_sh.at[idx_v.at[pl.ds(k * CH, CH)]],
                rows_v.at[pl.ds(k * CH, CH)],
                gsem,
            )
            for k in range(NCHUNK)
        ]
        writes = []
        for k in range(NCHUNK):
            gathers[k].wait()
            writes.append(
                pltpu.async_copy(
                    rows_v.at[pl.ds(k * CH, CH)],
                    out_hbm.at[pl.ds(base + k * CH, CH)],
                    wsem,
                )
            )
        for w in writes:
            w.wait()

    return lookup_kernel(table, indices.astype(jnp.int32))


# speedup vs baseline: 2.3810x; 1.0164x over previous
import functools

import jax
import jax.numpy as jnp
from jax import lax
from jax.experimental import pallas as pl
from jax.experimental.pallas import tpu as pltpu
from jax.experimental.pallas import tpu_sc as plsc

B = 16384
D = 128
V = 10
NC = 2
NS = 16
NW = NC * NS
B_PER_W = B // NW
NCHUNK = 8
CH = B_PER_W // NCHUNK


@jax.jit
def kernel(indices, table):
    mesh = plsc.VectorSubcoreMesh(core_axis_name="c", subcore_axis_name="s")

    @functools.partial(
        pl.kernel,
        mesh=mesh,
        out_type=jax.ShapeDtypeStruct((B, D), jnp.float32),
        scratch_types=[
            pltpu.VMEM((B_PER_W,), jnp.int32),
            pltpu.VMEM_SHARED((V, D), jnp.float32),
            pltpu.VMEM((B_PER_W, D), jnp.float32),
            pltpu.SemaphoreType.DMA,
            pltpu.SemaphoreType.DMA,
        ],
    )
    def lookup_kernel(table_hbm, idx_hbm, out_hbm, idx_v, tbl_sh, rows_v,
                      gsem, wsem):
        sid = lax.axis_index("s")
        wid = sid * NC + lax.axis_index("c")
        base = wid * B_PER_W

        idx_cp = pltpu.async_copy(idx_hbm.at[pl.ds(base, B_PER_W)], idx_v,
                                  wsem)

        @pl.when(sid == 0)
        def _():
            pltpu.sync_copy(table_hbm, tbl_sh)

        idx_cp.wait()
        plsc.subcore_barrier()

        gathers = [
            pltpu.async_copy(
                tbl_sh.at[idx_v.at[pl.ds(k * CH, CH)]],
                rows_v.at[pl.ds(k * CH, CH)],
                gsem,
            )
            for k in range(NCHUNK)
        ]
        writes = []
        for k in range(NCHUNK):
            gathers[k].wait()
            writes.append(
                pltpu.async_copy(
                    rows_v.at[pl.ds(k * CH, CH)],
                    out_hbm.at[pl.ds(base + k * CH, CH)],
                    wsem,
                )
            )
        for w in writes:
            w.wait()

    return lookup_kernel(table, indices.astype(jnp.int32))


# speedup vs baseline: 2.3947x; 1.0058x over previous
import functools

import jax
import jax.numpy as jnp
from jax import lax
from jax.experimental import pallas as pl
from jax.experimental.pallas import tpu as pltpu
from jax.experimental.pallas import tpu_sc as plsc

B = 16384
D = 128
V = 10
NC = 2
NS = 16
NW = NC * NS
B_PER_W = B // NW
NCHUNK = 8
CH = B_PER_W // NCHUNK


@jax.jit
def kernel(indices, table):
    mesh = plsc.VectorSubcoreMesh(core_axis_name="c", subcore_axis_name="s")

    @functools.partial(
        pl.kernel,
        mesh=mesh,
        out_type=jax.ShapeDtypeStruct((B, D), jnp.float32),
        scratch_types=[
            pltpu.VMEM((B_PER_W,), jnp.int32),
            pltpu.VMEM_SHARED((V, D), jnp.float32),
            pltpu.VMEM((B_PER_W, D), jnp.float32),
            pltpu.SemaphoreType.DMA,
            pltpu.SemaphoreType.DMA,
        ],
    )
    def lookup_kernel(table_hbm, idx_hbm, out_hbm, idx_v, tbl_sh, rows_v,
                      gsem, wsem):
        sid = lax.axis_index("s")
        wid = sid * NC + lax.axis_index("c")
        base = wid * B_PER_W

        idx_cps = [
            pltpu.async_copy(
                idx_hbm.at[pl.ds(base + k * CH, CH)],
                idx_v.at[pl.ds(k * CH, CH)],
                wsem,
            )
            for k in range(NCHUNK)
        ]

        @pl.when(sid == 0)
        def _():
            pltpu.sync_copy(table_hbm, tbl_sh)

        plsc.subcore_barrier()

        gathers = []
        for k in range(NCHUNK):
            idx_cps[k].wait()
            gathers.append(
                pltpu.async_copy(
                    tbl_sh.at[idx_v.at[pl.ds(k * CH, CH)]],
                    rows_v.at[pl.ds(k * CH, CH)],
                    gsem,
                )
            )
        writes = []
        for k in range(NCHUNK):
            gathers[k].wait()
            writes.append(
                pltpu.async_copy(
                    rows_v.at[pl.ds(k * CH, CH)],
                    out_hbm.at[pl.ds(base + k * CH, CH)],
                    wsem,
                )
            )
        for w in writes:
            w.wait()

    return lookup_kernel(table, indices.astype(jnp.int32))
